# baseline (speedup 1.0000x reference)
import functools

import jax
import jax.numpy as jnp
from jax import lax
from jax.experimental import pallas as pl
from jax.experimental.pallas import tpu as pltpu
from jax.experimental.pallas import tpu_sc as plsc

B = 4096
T = 50
D = 64
V = 100000
NUM_CORES = 2
NUM_SUBCORES = 16
NW = NUM_CORES * NUM_SUBCORES
SEGMENTS = ((0, 16), (16, 16), (32, 16), (48, 2))
SEG_MAX = max(n for _, n in SEGMENTS)
HB = B // 2
CW = 2048
V_MAIN = (V // CW) * CW
V_REM = V - V_MAIN

_mesh = plsc.VectorSubcoreMesh(core_axis_name="c", subcore_axis_name="s")


@functools.partial(
    pl.kernel,
    out_type=jax.ShapeDtypeStruct((T, D, B), jnp.float32),
    mesh=_mesh,
    scratch_types=[
        pltpu.VMEM((V,), jnp.int32),
        pltpu.VMEM((CW,), jnp.float32),
        pltpu.VMEM((CW,), jnp.float32),
        pltpu.VMEM((2, HB), jnp.int32),
        pltpu.VMEM((2, HB), jnp.float32),
        pltpu.VMEM((2, 2, HB), jnp.float32),
        pltpu.VMEM_SHARED((SEG_MAX, B), jnp.int32),
        pltpu.VMEM_SHARED((SEG_MAX, B), jnp.float32),
        pltpu.SemaphoreType.DMA,
        pltpu.SemaphoreType.DMA,
        pltpu.SemaphoreType.DMA,
        pltpu.SemaphoreType.DMA,
    ],
    compiler_params=pltpu.CompilerParams(use_tc_tiling_on_sc=True,
                                         needs_layout_passes=False),
)
def _tag_embedding(tags_hbm, probs_hbm, table_hbm, tail_hbm, out_hbm,
                   col_v, ta_v, tb_v, tg_v, pr_v, ob_v, stg_t, stg_p,
                   sem_in0, sem_in1, sem_st0, sem_st1):
    sem_in = (sem_in0, sem_in1)
    sem_st = (sem_st0, sem_st1)
    sid = lax.axis_index("s")
    wid = sid * NUM_CORES + lax.axis_index("c")
    ch_a = 2 * wid
    ch_b = 2 * wid + 1

    def pack_chunk(off, nelem):
        pltpu.sync_copy(table_hbm.at[ch_a, pl.ds(off, nelem)],
                        ta_v.at[pl.ds(0, nelem)])
        pltpu.sync_copy(table_hbm.at[ch_b, pl.ds(off, nelem)],
                        tb_v.at[pl.ds(0, nelem)])

        @plsc.parallel_loop(0, nelem, step=16, unroll=8)
        def _(i):
            va = ta_v[pl.ds(i, 16)]
            vb = tb_v[pl.ds(i, 16)]
            pk = plsc.pack(va, vb, format=plsc.PackFormat.INTERLEAVED)
            col_v[pl.ds(off + i, 16)] = plsc.bitcast(pk, jnp.int32)

    @pl.loop(0, V_MAIN, step=CW)
    def _(off):
        pack_chunk(pl.multiple_of(off, 128), CW)

    pack_chunk(V_MAIN, V_REM - 32)

    pltpu.sync_copy(tail_hbm.at[ch_a], ta_v.at[pl.ds(0, 128)])
    pltpu.sync_copy(tail_hbm.at[ch_b], tb_v.at[pl.ds(0, 128)])

    @plsc.parallel_loop(0, 128, step=16, unroll=4)
    def _(i):
        va = ta_v[pl.ds(i, 16)]
        vb = tb_v[pl.ds(i, 16)]
        pk = plsc.pack(va, vb, format=plsc.PackFormat.INTERLEAVED)
        col_v[pl.ds((V - 128) + i, 16)] = plsc.bitcast(pk, jnp.int32)

    def stage(base, n):
        nfull = n // 8
        rem = n - nfull * 8

        @pl.when(sid < nfull)
        def _():
            src = pl.ds(base + sid * 8, 8)
            dst = pl.ds(sid * 8, 8)
            pltpu.sync_copy(tags_hbm.at[src], stg_t.at[dst])
            pltpu.sync_copy(probs_hbm.at[src], stg_p.at[dst])

        if rem:
            @pl.when(sid == nfull)
            def _():
                src = pl.ds(base + nfull * 8, rem)
                dst = pl.ds(nfull * 8, rem)
                pltpu.sync_copy(tags_hbm.at[src], stg_t.at[dst])
                pltpu.sync_copy(probs_hbm.at[src], stg_p.at[dst])

    def in_refs(j, s):
        t = j >> 1
        hsl = pl.ds((j & 1) * HB, HB)
        return ((stg_t.at[t, hsl], tg_v.at[s]),
                (stg_p.at[t, hsl], pr_v.at[s]))

    def load_in(j, s):
        for src, dst in in_refs(j, s):
            pltpu.async_copy(src, dst, sem_in[s])

    def wait_in(j, s):
        for src, dst in in_refs(j, s):
            pltpu.make_async_copy(src, dst, sem_in[s]).wait()

    def out_refs(base, j, s):
        t = base + (j >> 1)
        hsl = pl.ds((j & 1) * HB, HB)
        return ((ob_v.at[s, 0], out_hbm.at[t, ch_a, hsl]),
                (ob_v.at[s, 1], out_hbm.at[t, ch_b, hsl]))

    def store_out(base, j, s):
        for src, dst in out_refs(base, j, s):
            pltpu.async_copy(src, dst, sem_st[s])

    def wait_out(base, j, s):
        for src, dst in out_refs(base, j, s):
            pltpu.make_async_copy(src, dst, sem_st[s]).wait()

    def compute(s):
        @plsc.parallel_loop(0, HB, step=16, unroll=4)
        def _(i):
            sl = pl.ds(i, 16)
            idx = tg_v[s, sl]
            pki = plsc.load_gather(col_v, [idx])
            pkb = plsc.bitcast(pki, jnp.bfloat16)
            va, vb = plsc.unpack(pkb, format=plsc.PackFormat.INTERLEAVED)
            pv = pr_v[s, sl]
            ob_v[s, 0, sl] = va * pv
            ob_v[s, 1, sl] = vb * pv

    for base, n in SEGMENTS:
        nj = 2 * n
        plsc.subcore_barrier()
        stage(base, n)
        plsc.subcore_barrier()

        load_in(0, 0)
        load_in(1, 1)

        @pl.loop(0, nj, step=2)
        def t_loop(g):
            for s in range(2):
                j = g + s
                wait_in(j, s)

                @pl.when(j >= 2)
                def _():
                    wait_out(base, j - 2, s)

                compute(s)
                store_out(base, j, s)

                @pl.when(j + 2 < nj)
                def _():
                    load_in(j + 2, s)

        wait_out(base, nj - 2, 0)
        wait_out(base, nj - 1, 1)


def kernel(tags, probs, table):
    table_t = table.T
    out_t = _tag_embedding(tags.T.astype(jnp.int32), probs.T, table_t,
                           table_t[:, V - 128:])
    return jnp.transpose(out_t, (2, 0, 1))


# speedup vs baseline: 1.2883x; 1.2883x over previous
import functools

import jax
import jax.numpy as jnp
from jax import lax
from jax.experimental import pallas as pl
from jax.experimental.pallas import tpu as pltpu
from jax.experimental.pallas import tpu_sc as plsc

B = 4096
T = 50
D = 64
V = 100000
NUM_CORES = 2
NUM_SUBCORES = 16
NW = NUM_CORES * NUM_SUBCORES
SEGMENTS = ((0, 8), (8, 8), (16, 8), (24, 8), (32, 8), (40, 8), (48, 2))
SEG_MAX = max(n for _, n in SEGMENTS)
HB = B // 2
CW = 2048
V_MAIN = (V // CW) * CW
V_REM = V - V_MAIN

_mesh = plsc.VectorSubcoreMesh(core_axis_name="c", subcore_axis_name="s")


@functools.partial(
    pl.kernel,
    out_type=jax.ShapeDtypeStruct((T, D, B), jnp.float32),
    mesh=_mesh,
    scratch_types=[
        pltpu.VMEM((V,), jnp.int32),
        pltpu.VMEM((2, CW), jnp.float32),
        pltpu.VMEM((2, CW), jnp.float32),
        pltpu.VMEM((2, HB), jnp.int32),
        pltpu.VMEM((2, HB), jnp.float32),
        pltpu.VMEM((2, 2, HB), jnp.float32),
        pltpu.VMEM_SHARED((SEG_MAX, B), jnp.int32),
        pltpu.VMEM_SHARED((SEG_MAX, B), jnp.float32),
        pltpu.SemaphoreType.DMA,
        pltpu.SemaphoreType.DMA,
        pltpu.SemaphoreType.DMA,
        pltpu.SemaphoreType.DMA,
    ],
    compiler_params=pltpu.CompilerParams(use_tc_tiling_on_sc=True,
                                         needs_layout_passes=False),
)
def _tag_embedding(tags_hbm, probs_hbm, table_hbm, tail_hbm, out_hbm,
                   col_v, ta_v, tb_v, tg_v, pr_v, ob_v, stg_t, stg_p,
                   sem_in0, sem_in1, sem_st0, sem_st1):
    sem_in = (sem_in0, sem_in1)
    sem_st = (sem_st0, sem_st1)
    sid = lax.axis_index("s")
    wid = sid * NUM_CORES + lax.axis_index("c")
    ch_a = 2 * wid
    ch_b = 2 * wid + 1

    def pack_region(slot, dst_off, nelem, unroll=8):
        @plsc.parallel_loop(0, nelem, step=16, unroll=unroll)
        def _(i):
            va = ta_v[slot, pl.ds(i, 16)]
            vb = tb_v[slot, pl.ds(i, 16)]
            pk = plsc.pack(va, vb, format=plsc.PackFormat.INTERLEAVED)
            col_v[pl.ds(dst_off + i, 16)] = plsc.bitcast(pk, jnp.int32)

    def chunk_refs(off, s):
        return ((table_hbm.at[pl.ds(ch_a, 1), pl.ds(off, CW)],
                 ta_v.at[pl.ds(s, 1)]),
                (table_hbm.at[pl.ds(ch_b, 1), pl.ds(off, CW)],
                 tb_v.at[pl.ds(s, 1)]))

    def load_chunk(off, s):
        for src, dst in chunk_refs(off, s):
            pltpu.async_copy(src, dst, sem_in[s])

    def wait_chunk(off, s):
        for src, dst in chunk_refs(off, s):
            pltpu.make_async_copy(src, dst, sem_in[s]).wait()

    load_chunk(0, 0)
    load_chunk(CW, 1)

    @pl.loop(0, V_MAIN, step=2 * CW)
    def _(off0):
        for s in range(2):
            off = pl.multiple_of(off0, 128) + s * CW
            wait_chunk(off, s)
            pack_region(s, off, CW)

            @pl.when(off + 2 * CW < V_MAIN)
            def _():
                load_chunk(off + 2 * CW, s)

    pltpu.sync_copy(table_hbm.at[pl.ds(ch_a, 1), pl.ds(V_MAIN, V_REM - 32)],
                    ta_v.at[pl.ds(0, 1), pl.ds(0, V_REM - 32)])
    pltpu.sync_copy(table_hbm.at[pl.ds(ch_b, 1), pl.ds(V_MAIN, V_REM - 32)],
                    tb_v.at[pl.ds(0, 1), pl.ds(0, V_REM - 32)])
    pack_region(0, V_MAIN, V_REM - 32)

    pltpu.sync_copy(tail_hbm.at[pl.ds(ch_a, 1)], ta_v.at[pl.ds(0, 1), pl.ds(0, 128)])
    pltpu.sync_copy(tail_hbm.at[pl.ds(ch_b, 1)], tb_v.at[pl.ds(0, 1), pl.ds(0, 128)])
    pack_region(0, V - 128, 128, unroll=4)

    def stage(base, n):
        nfull = n // 8
        rem = n - nfull * 8

        @pl.when(sid < nfull)
        def _():
            src = pl.ds(base + sid * 8, 8)
            dst = pl.ds(sid * 8, 8)
            pltpu.sync_copy(tags_hbm.at[src], stg_t.at[dst])
            pltpu.sync_copy(probs_hbm.at[src], stg_p.at[dst])

        if rem:
            @pl.when(sid == nfull)
            def _():
                src = pl.ds(base + nfull * 8, rem)
                dst = pl.ds(nfull * 8, rem)
                pltpu.sync_copy(tags_hbm.at[src], stg_t.at[dst])
                pltpu.sync_copy(probs_hbm.at[src], stg_p.at[dst])

    def in_refs(j, s):
        t = j >> 1
        hsl = pl.ds((j & 1) * HB, HB)
        return ((stg_t.at[t, hsl], tg_v.at[s]),
                (stg_p.at[t, hsl], pr_v.at[s]))

    def load_in(j, s):
        for src, dst in in_refs(j, s):
            pltpu.async_copy(src, dst, sem_in[s])

    def wait_in(j, s):
        for src, dst in in_refs(j, s):
            pltpu.make_async_copy(src, dst, sem_in[s]).wait()

    def out_refs(base, j, s):
        t = base + (j >> 1)
        hsl = pl.ds((j & 1) * HB, HB)
        return ((ob_v.at[s, 0], out_hbm.at[t, ch_a, hsl]),
                (ob_v.at[s, 1], out_hbm.at[t, ch_b, hsl]))

    def store_out(base, j, s):
        for src, dst in out_refs(base, j, s):
            pltpu.async_copy(src, dst, sem_st[s])

    def wait_out(base, j, s):
        for src, dst in out_refs(base, j, s):
            pltpu.make_async_copy(src, dst, sem_st[s]).wait()

    def compute(s):
        @plsc.parallel_loop(0, HB, step=16, unroll=4)
        def _(i):
            sl = pl.ds(i, 16)
            idx = tg_v[s, sl]
            pki = plsc.load_gather(col_v, [idx])
            pkb = plsc.bitcast(pki, jnp.bfloat16)
            va, vb = plsc.unpack(pkb, format=plsc.PackFormat.INTERLEAVED)
            pv = pr_v[s, sl]
            ob_v[s, 0, sl] = va * pv
            ob_v[s, 1, sl] = vb * pv

    for base, n in SEGMENTS:
        nj = 2 * n
        plsc.subcore_barrier()
        stage(base, n)
        plsc.subcore_barrier()

        load_in(0, 0)
        load_in(1, 1)

        @pl.loop(0, nj, step=2)
        def t_loop(g):
            for s in range(2):
                j = g + s
                wait_in(j, s)

                @pl.when(j >= 2)
                def _():
                    wait_out(base, j - 2, s)

                compute(s)
                store_out(base, j, s)

                @pl.when(j + 2 < nj)
                def _():
                    load_in(j + 2, s)

        wait_out(base, nj - 2, 0)
        wait_out(base, nj - 1, 1)


def kernel(tags, probs, table):
    table_t = table.T
    out_t = _tag_embedding(tags.T.astype(jnp.int32), probs.T, table_t,
                           table_t[:, V - 128:])
    return jnp.transpose(out_t, (2, 0, 1))


# speedup vs baseline: 1.2923x; 1.0031x over previous
import functools

import jax
import jax.numpy as jnp
from jax import lax
from jax.experimental import pallas as pl
from jax.experimental.pallas import tpu as pltpu
from jax.experimental.pallas import tpu_sc as plsc

B = 4096
T = 50
D = 64
V = 100000
NUM_CORES = 2
NUM_SUBCORES = 16
NW = NUM_CORES * NUM_SUBCORES
SEGMENTS = ((0, 8), (8, 8), (16, 8), (24, 8), (32, 8), (40, 8), (48, 2))
SEG_MAX = max(n for _, n in SEGMENTS)
HB = B // 2
CW = 2048
V_MAIN = (V // CW) * CW
V_REM = V - V_MAIN

_mesh = plsc.VectorSubcoreMesh(core_axis_name="c", subcore_axis_name="s")


@functools.partial(
    pl.kernel,
    out_type=jax.ShapeDtypeStruct((T, D, B), jnp.float32),
    mesh=_mesh,
    scratch_types=[
        pltpu.VMEM((V,), jnp.int32),
        pltpu.VMEM((2, CW), jnp.float32),
        pltpu.VMEM((2, CW), jnp.float32),
        pltpu.VMEM((2, HB), jnp.int32),
        pltpu.VMEM((2, HB), jnp.float32),
        pltpu.VMEM((2, 2, HB), jnp.float32),
        pltpu.VMEM_SHARED((SEG_MAX, B), jnp.int32),
        pltpu.VMEM_SHARED((SEG_MAX, B), jnp.float32),
        pltpu.SemaphoreType.DMA,
        pltpu.SemaphoreType.DMA,
        pltpu.SemaphoreType.DMA,
        pltpu.SemaphoreType.DMA,
    ],
    compiler_params=pltpu.CompilerParams(use_tc_tiling_on_sc=True,
                                         needs_layout_passes=False),
)
def _tag_embedding(tags_hbm, probs_hbm, table_hbm, tail_hbm, out_hbm,
                   col_v, ta_v, tb_v, tg_v, pr_v, ob_v, stg_t, stg_p,
                   sem_in0, sem_in1, sem_st0, sem_st1):
    sem_in = (sem_in0, sem_in1)
    sem_st = (sem_st0, sem_st1)
    sid = lax.axis_index("s")
    wid = sid * NUM_CORES + lax.axis_index("c")
    ch_a = 2 * wid
    ch_b = 2 * wid + 1

    def pack_region(slot, dst_off, nelem, unroll=8):
        @plsc.parallel_loop(0, nelem, step=16, unroll=unroll)
        def _(i):
            va = ta_v[slot, pl.ds(i, 16)]
            vb = tb_v[slot, pl.ds(i, 16)]
            pk = plsc.pack(va, vb, format=plsc.PackFormat.INTERLEAVED)
            col_v[pl.ds(dst_off + i, 16)] = plsc.bitcast(pk, jnp.int32)

    def chunk_refs(off, s):
        return ((table_hbm.at[pl.ds(ch_a, 1), pl.ds(off, CW)],
                 ta_v.at[pl.ds(s, 1)]),
                (table_hbm.at[pl.ds(ch_b, 1), pl.ds(off, CW)],
                 tb_v.at[pl.ds(s, 1)]))

    def load_chunk(off, s):
        for src, dst in chunk_refs(off, s):
            pltpu.async_copy(src, dst, sem_in[s])

    def wait_chunk(off, s):
        for src, dst in chunk_refs(off, s):
            pltpu.make_async_copy(src, dst, sem_in[s]).wait()

    load_chunk(0, 0)
    load_chunk(CW, 1)

    @pl.loop(0, V_MAIN, step=2 * CW)
    def _(off0):
        for s in range(2):
            off = pl.multiple_of(off0, 128) + s * CW
            wait_chunk(off, s)
            pack_region(s, off, CW)

            @pl.when(off + 2 * CW < V_MAIN)
            def _():
                load_chunk(off + 2 * CW, s)

    pltpu.sync_copy(table_hbm.at[pl.ds(ch_a, 1), pl.ds(V_MAIN, V_REM - 32)],
                    ta_v.at[pl.ds(0, 1), pl.ds(0, V_REM - 32)])
    pltpu.sync_copy(table_hbm.at[pl.ds(ch_b, 1), pl.ds(V_MAIN, V_REM - 32)],
                    tb_v.at[pl.ds(0, 1), pl.ds(0, V_REM - 32)])
    pack_region(0, V_MAIN, V_REM - 32)

    pltpu.sync_copy(tail_hbm.at[pl.ds(ch_a, 1)], ta_v.at[pl.ds(0, 1), pl.ds(0, 128)])
    pltpu.sync_copy(tail_hbm.at[pl.ds(ch_b, 1)], tb_v.at[pl.ds(0, 1), pl.ds(0, 128)])
    pack_region(0, V - 128, 128, unroll=4)

    def stage(base, n):
        nfull = n // 8
        rem = n - nfull * 8

        @pl.when(sid < nfull)
        def _():
            src = pl.ds(base + sid * 8, 8)
            dst = pl.ds(sid * 8, 8)
            pltpu.sync_copy(tags_hbm.at[src], stg_t.at[dst])
            pltpu.sync_copy(probs_hbm.at[src], stg_p.at[dst])

        if rem:
            @pl.when(sid == nfull)
            def _():
                src = pl.ds(base + nfull * 8, rem)
                dst = pl.ds(nfull * 8, rem)
                pltpu.sync_copy(tags_hbm.at[src], stg_t.at[dst])
                pltpu.sync_copy(probs_hbm.at[src], stg_p.at[dst])

    def in_refs(j, s):
        t = j >> 1
        hsl = pl.ds((j & 1) * HB, HB)
        return ((stg_t.at[t, hsl], tg_v.at[s]),
                (stg_p.at[t, hsl], pr_v.at[s]))

    def load_in(j, s):
        for src, dst in in_refs(j, s):
            pltpu.async_copy(src, dst, sem_in[s])

    def wait_in(j, s):
        for src, dst in in_refs(j, s):
            pltpu.make_async_copy(src, dst, sem_in[s]).wait()

    def out_refs(base, j, s):
        t = base + (j >> 1)
        hsl = pl.ds((j & 1) * HB, HB)
        return ((ob_v.at[s, 0], out_hbm.at[t, ch_a, hsl]),
                (ob_v.at[s, 1], out_hbm.at[t, ch_b, hsl]))

    def store_out(base, j, s):
        for src, dst in out_refs(base, j, s):
            pltpu.async_copy(src, dst, sem_st[s])

    def wait_out(base, j, s):
        for src, dst in out_refs(base, j, s):
            pltpu.make_async_copy(src, dst, sem_st[s]).wait()

    def compute(s):
        @plsc.parallel_loop(0, HB, step=16, unroll=8)
        def _(i):
            sl = pl.ds(i, 16)
            idx = tg_v[s, sl]
            pki = plsc.load_gather(col_v, [idx])
            pkb = plsc.bitcast(pki, jnp.bfloat16)
            va, vb = plsc.unpack(pkb, format=plsc.PackFormat.INTERLEAVED)
            pv = pr_v[s, sl]
            ob_v[s, 0, sl] = va * pv
            ob_v[s, 1, sl] = vb * pv

    for base, n in SEGMENTS:
        nj = 2 * n
        plsc.subcore_barrier()
        stage(base, n)
        plsc.subcore_barrier()

        load_in(0, 0)
        load_in(1, 1)

        @pl.loop(0, nj, step=2)
        def t_loop(g):
            for s in range(2):
                j = g + s
                wait_in(j, s)

                @pl.when(j >= 2)
                def _():
                    wait_out(base, j - 2, s)

                compute(s)
                store_out(base, j, s)

                @pl.when(j + 2 < nj)
                def _():
                    load_in(j + 2, s)

        wait_out(base, nj - 2, 0)
        wait_out(base, nj - 1, 1)


def kernel(tags, probs, table):
    table_t = table.T
    out_t = _tag_embedding(tags.T.astype(jnp.int32), probs.T, table_t,
                           table_t[:, V - 128:])
    return jnp.transpose(out_t, (2, 0, 1))
